# Initial kernel scaffold; baseline (speedup 1.0000x reference)
import jax
import jax.numpy as jnp
from jax.experimental import pallas as pl


def kernel(af, bf, fnf, fef, atom_edge_index, motif_edge_index, a2f_index, atom_batch, motif_batch, params):
    raise NotImplementedError("write your pallas kernel here")


# speedup vs baseline: 1.1935x; 1.1935x over previous
import functools

import jax
import jax.numpy as jnp
from jax import lax
from jax.experimental import pallas as pl
from jax.experimental.pallas import tpu as pltpu
from jax.experimental.pallas import tpu_sc as plsc

HID = 16
HEADS = 4
G = 64
NC, NS, L = 2, 16, 16
NW = NC * NS
CHUNK = 128

N_ATOM, E_ATOM = 10000, 160000
N_MOTIF, E_MOTIF = 2000, 8000

EA_PAD = 163840
EM_PAD = 8192
NA_BUF = 12288
NM_BUF = 2048

CA = EA_PAD // NW // CHUNK
CM = EM_PAD // NW // CHUNK
CF = NA_BUF // NS // CHUNK

_F32 = jnp.float32


def _lbr(x, W, b):
    h = jnp.dot(x, W, preferred_element_type=_F32, precision=lax.Precision.HIGHEST) + b
    mu = jnp.mean(h, axis=0, keepdims=True)
    var = jnp.mean(h * h, axis=0, keepdims=True) - mu * mu
    h = (h - mu) * lax.rsqrt(var + 1e-5)
    return jnp.maximum(h, 0.0)


def _encoder_body(af, fnf, aW1, ab1, aW2, ab2, mW1, mb1, mW2, mb2,
                  uaf_o, ufnf_o):
    uaf_o[...] = _lbr(_lbr(af[...], aW1[...], ab1[...]), aW2[...], ab2[...])
    ufnf_o[...] = _lbr(_lbr(fnf[...], mW1[...], mb1[...]), mW2[...], mb2[...])


def _encoders(af, fnf, p):
    return pl.pallas_call(
        _encoder_body,
        out_shape=[jax.ShapeDtypeStruct((N_ATOM, HID), _F32),
                   jax.ShapeDtypeStruct((N_MOTIF, HID), _F32)],
    )(af, fnf,
      p['ae_W1'], p['ae_b1'].reshape(1, HID), p['ae_W2'], p['ae_b2'].reshape(1, HID),
      p['me_W1'], p['me_b1'].reshape(1, HID), p['me_W2'], p['me_b2'].reshape(1, HID))


def _gather_body(srcA, srcM, uaf, ufnf, outA, outM,
                 idxA_v, rowsA_v, idxM_v, rowsM_v, sem):
    s = lax.axis_index("s")
    c = lax.axis_index("c")
    wid = s * NC + c

    pltpu.sync_copy(srcA.at[pl.ds(wid * CA, CA)], idxA_v)
    pltpu.sync_copy(srcM.at[pl.ds(wid * CM, CM)], idxM_v)

    def fire8(i, _):
        descs = []
        for b in range(8):
            j = i * 8 + b
            descs.append(pltpu.async_copy(
                uaf.at[idxA_v.at[j]], rowsA_v.at[pl.ds(j * CHUNK, CHUNK)], sem))
        for d in descs:
            d.wait()
        return 0

    lax.fori_loop(0, CA // 8, fire8, 0)

    descs = []
    for j in range(CM):
        descs.append(pltpu.async_copy(
            ufnf.at[idxM_v.at[j]], rowsM_v.at[pl.ds(j * CHUNK, CHUNK)], sem))
    for d in descs:
        d.wait()

    pltpu.sync_copy(rowsA_v, outA.at[pl.ds(wid * CA * CHUNK, CA * CHUNK)])
    pltpu.sync_copy(rowsM_v, outM.at[pl.ds(wid * CM * CHUNK, CM * CHUNK)])


def _sc_gather(uaf, ufnf, srcA2d, srcM2d):
    k = functools.partial(
        pl.kernel,
        out_type=[jax.ShapeDtypeStruct((EA_PAD, HID), _F32),
                  jax.ShapeDtypeStruct((EM_PAD, HID), _F32)],
        mesh=plsc.VectorSubcoreMesh(core_axis_name="c", subcore_axis_name="s"),
        scratch_types=[
            pltpu.VMEM((CA, CHUNK), jnp.int32),
            pltpu.VMEM((CA * CHUNK, HID), _F32),
            pltpu.VMEM((CM, CHUNK), jnp.int32),
            pltpu.VMEM((CM * CHUNK, HID), _F32),
            pltpu.SemaphoreType.DMA,
        ],
        compiler_params=pltpu.CompilerParams(use_tc_tiling_on_sc=False),
    )(_gather_body)
    return k(srcA2d, srcM2d, uaf, ufnf)


def _msg_body(ef, xg, We, be, out):
    P = jnp.dot(ef[...], We[...], preferred_element_type=_F32, precision=lax.Precision.HIGHEST) + be[...]
    xgv = xg[...]
    acc = xgv[:, 0:1] * P[:, 0:HID]
    for i in range(1, HID):
        acc = acc + xgv[:, i:i + 1] * P[:, i * HID:(i + 1) * HID]
    out[...] = acc


def _edge_messages(ef, xg, We, be, blk):
    e_pad, fdim = ef.shape
    grid = e_pad // blk
    return pl.pallas_call(
        _msg_body,
        grid=(grid,),
        in_specs=[
            pl.BlockSpec((blk, fdim), lambda i: (i, 0)),
            pl.BlockSpec((blk, HID), lambda i: (i, 0)),
            pl.BlockSpec((fdim, HID * HID), lambda i: (0, 0)),
            pl.BlockSpec((1, HID * HID), lambda i: (0, 0)),
        ],
        out_specs=pl.BlockSpec((blk, HID), lambda i: (i, 0)),
        out_shape=jax.ShapeDtypeStruct((e_pad, HID), _F32),
    )(ef, xg, We, be.reshape(1, HID * HID))


def _scatter_body(msgA, msgM, dstA, dstM, a2f,
                  aggA_o, aggM_o, aggF_o, cnt_o,
                  mbuf, idxA_v, idxM_v, idxF_v, stage_v, ones_v,
                  aggA_sh, aggM_sh, aggF_sh, cnt_sh, sem):
    s = lax.axis_index("s")
    c = lax.axis_index("c")
    wid = s * NC + c
    rows_a = NA_BUF // NS
    rows_m = NM_BUF // NS

    def zrow(i, _):
        stage_v[i] = jnp.zeros((L,), _F32)
        return 0
    lax.fori_loop(0, rows_a, zrow, 0)

    def orow(i, _):
        ones_v[i] = jnp.ones((L,), _F32)
        return 0
    lax.fori_loop(0, CHUNK, orow, 0)

    pltpu.sync_copy(stage_v, aggA_sh.at[pl.ds(s * rows_a, rows_a)])
    pltpu.sync_copy(stage_v.at[pl.ds(0, rows_m)], aggM_sh.at[pl.ds(s * rows_m, rows_m)])
    pltpu.sync_copy(stage_v.at[pl.ds(0, rows_m)], aggF_sh.at[pl.ds(s * rows_m, rows_m)])
    pltpu.sync_copy(stage_v.at[pl.ds(0, rows_m)], cnt_sh.at[pl.ds(s * rows_m, rows_m)])
    plsc.subcore_barrier()

    pltpu.sync_copy(dstA.at[pl.ds(wid * CA, CA)], idxA_v)
    pltpu.sync_copy(msgA.at[pl.ds(wid * CA * CHUNK, CA * CHUNK)], mbuf)

    def fire8(i, _):
        descs = []
        for b in range(8):
            j = i * 8 + b
            descs.append(pltpu.async_copy(
                mbuf.at[pl.ds(j * CHUNK, CHUNK)], aggA_sh.at[idxA_v.at[j]],
                sem, add=True))
        for d in descs:
            d.wait()
        return 0
    lax.fori_loop(0, CA // 8, fire8, 0)

    pltpu.sync_copy(dstM.at[pl.ds(wid * CM, CM)], idxM_v)
    pltpu.sync_copy(msgM.at[pl.ds(wid * CM * CHUNK, CM * CHUNK)],
                    mbuf.at[pl.ds(0, CM * CHUNK)])
    descs = []
    for j in range(CM):
        descs.append(pltpu.async_copy(
            mbuf.at[pl.ds(j * CHUNK, CHUNK)], aggM_sh.at[idxM_v.at[j]],
            sem, add=True))
    for d in descs:
        d.wait()
    plsc.subcore_barrier()

    pltpu.sync_copy(aggA_sh.at[pl.ds(s * rows_a, rows_a)], stage_v)
    pltpu.sync_copy(a2f.at[pl.ds(s * CF, CF)], idxF_v)
    descs = []
    for j in range(CF):
        descs.append(pltpu.async_copy(
            stage_v.at[pl.ds(j * CHUNK, CHUNK)], aggF_sh.at[idxF_v.at[j]],
            sem, add=True))
        descs.append(pltpu.async_copy(
            ones_v, cnt_sh.at[idxF_v.at[j]], sem, add=True))
    for d in descs:
        d.wait()
    plsc.subcore_barrier()

    pltpu.sync_copy(aggA_sh.at[pl.ds(s * rows_a, rows_a)],
                    aggA_o.at[c, pl.ds(s * rows_a, rows_a)])
    pltpu.sync_copy(aggM_sh.at[pl.ds(s * rows_m, rows_m)],
                    aggM_o.at[c, pl.ds(s * rows_m, rows_m)])
    pltpu.sync_copy(aggF_sh.at[pl.ds(s * rows_m, rows_m)],
                    aggF_o.at[c, pl.ds(s * rows_m, rows_m)])
    pltpu.sync_copy(cnt_sh.at[pl.ds(s * rows_m, rows_m)],
                    cnt_o.at[c, pl.ds(s * rows_m, rows_m)])


def _sc_scatter(msgA, msgM, dstA2d, dstM2d, a2f2d):
    k = functools.partial(
        pl.kernel,
        out_type=[jax.ShapeDtypeStruct((NC, NA_BUF, HID), _F32),
                  jax.ShapeDtypeStruct((NC, NM_BUF, HID), _F32),
                  jax.ShapeDtypeStruct((NC, NM_BUF, HID), _F32),
                  jax.ShapeDtypeStruct((NC, NM_BUF, HID), _F32)],
        mesh=plsc.VectorSubcoreMesh(core_axis_name="c", subcore_axis_name="s"),
        scratch_types=[
            pltpu.VMEM((CA * CHUNK, HID), _F32),
            pltpu.VMEM((CA, CHUNK), jnp.int32),
            pltpu.VMEM((CM, CHUNK), jnp.int32),
            pltpu.VMEM((CF, CHUNK), jnp.int32),
            pltpu.VMEM((NA_BUF // NS, HID), _F32),
            pltpu.VMEM((CHUNK, HID), _F32),
            pltpu.VMEM_SHARED((NA_BUF, HID), _F32),
            pltpu.VMEM_SHARED((NM_BUF, HID), _F32),
            pltpu.VMEM_SHARED((NM_BUF, HID), _F32),
            pltpu.VMEM_SHARED((NM_BUF, HID), _F32),
            pltpu.SemaphoreType.DMA,
        ],
        compiler_params=pltpu.CompilerParams(use_tc_tiling_on_sc=False),
    )(_scatter_body)
    return k(msgA, msgM, dstA2d, dstM2d, a2f2d)


def _sigmoid(x):
    return 1.0 / (1.0 + jnp.exp(-x))


def _gru(x, h, Wih, Whh, bih, bhh):
    gi = jnp.dot(x, Wih, preferred_element_type=_F32) + bih
    gh = jnp.dot(h, Whh, preferred_element_type=_F32) + bhh
    r = _sigmoid(gi[:, 0:HID] + gh[:, 0:HID])
    z = _sigmoid(gi[:, HID:2 * HID] + gh[:, HID:2 * HID])
    n = jnp.tanh(gi[:, 2 * HID:] + r * gh[:, 2 * HID:])
    return (1.0 - z) * n + z * h


def _fuse_body(uaf, ufnf, aggA2, aggM2, aggF2, cnt2, ab, mb,
               ac_bias, mc_bias, Wq, Wk, Wv, Wo, bo,
               gmWih, gmWhh, gmbih, gmbhh, gaWih, gaWhh, gabih, gabhh,
               cpW1, cpb1, cpW2, cpb2, cmW1, cmb1, cmW2, cmb2, cqW, cqb,
               out, comb_ref):
    uam = aggA2[0, :N_ATOM, :] + aggA2[1, :N_ATOM, :] + ac_bias[...]
    ufnm = aggM2[0, :N_MOTIF, :] + aggM2[1, :N_MOTIF, :] + mc_bias[...]
    agg_uam = (aggF2[0, :N_MOTIF, :] + aggF2[1, :N_MOTIF, :]
               + cnt2[0, :N_MOTIF, :] * ac_bias[...])

    ufnf_v = ufnf[...]
    uaf_v = uaf[...]

    dk = HID // HEADS
    ii = lax.broadcasted_iota(jnp.int32, (HID, HEADS), 0) // dk
    hh = lax.broadcasted_iota(jnp.int32, (HID, HEADS), 1)
    GH = (ii == hh).astype(_F32)
    GHT = GH.T

    Q = jnp.dot(ufnf_v, Wq[...], preferred_element_type=_F32)
    Kf = jnp.dot(agg_uam, Wk[...], preferred_element_type=_F32)
    Kc = jnp.dot(ufnm, Wk[...], preferred_element_type=_F32)
    Vf = jnp.dot(agg_uam, Wv[...], preferred_element_type=_F32)
    Vc = jnp.dot(ufnm, Wv[...], preferred_element_type=_F32)
    s0 = jnp.dot(Q * Kf, GH, preferred_element_type=_F32) / dk
    s1 = jnp.dot(Q * Kc, GH, preferred_element_type=_F32) / dk
    m = jnp.maximum(s0, s1)
    e0 = jnp.exp(s0 - m)
    e1 = jnp.exp(s1 - m)
    w0 = e0 / (e0 + e1)
    w1 = 1.0 - w0
    att = (jnp.dot(w0, GHT, preferred_element_type=_F32) * Vf
           + jnp.dot(w1, GHT, preferred_element_type=_F32) * Vc)
    motif_msg = jnp.dot(att, Wo[...], preferred_element_type=_F32) + bo[...]

    ufnf_n = _gru(motif_msg, ufnf_v, gmWih[...], gmWhh[...], gmbih[...], gmbhh[...])
    uaf_n = _gru(uam, uaf_v, gaWih[...], gaWhh[...], gabih[...], gabhh[...])

    ab_v = ab[...]
    mb_v = mb[...]
    neg = jnp.float32(-3.0e38)

    def rd(g, _):
        amask = (ab_v == g)
        af32 = amask.astype(_F32)
        acnt = jnp.sum(af32)
        asum = jnp.sum(af32 * uaf_n, axis=0, keepdims=True)
        amean = asum / jnp.maximum(acnt, 1.0)
        amax = jnp.max(jnp.where(amask, uaf_n, neg), axis=0, keepdims=True)
        amax = jnp.where(acnt > 0, amax, 0.0)

        mmask = (mb_v == g)
        mf32 = mmask.astype(_F32)
        mcnt = jnp.sum(mf32)
        msum = jnp.sum(mf32 * ufnf_n, axis=0, keepdims=True)
        mmean = msum / jnp.maximum(mcnt, 1.0)
        mmax = jnp.max(jnp.where(mmask, ufnf_n, neg), axis=0, keepdims=True)
        mmax = jnp.where(mcnt > 0, mmax, 0.0)

        comb_ref[pl.ds(g, 1), :] = jnp.concatenate(
            [amean, amax, mmean, mmax], axis=1)
        return 0

    lax.fori_loop(0, G, rd, 0)

    comb = comb_ref[...]
    rep = jnp.maximum(jnp.dot(comb, cpW1[...], preferred_element_type=_F32)
                      + cpb1[...], 0.0)
    rep = jnp.dot(rep, cpW2[...], preferred_element_type=_F32) + cpb2[...]
    h3 = jnp.maximum(jnp.dot(rep, cmW1[...], preferred_element_type=_F32)
                     + cmb1[...], 0.0)
    lg = jnp.dot(h3, cmW2[...], preferred_element_type=_F32) + cmb2[...]
    out[...] = jnp.dot(lg, cqW[...], preferred_element_type=_F32) + cqb[...]


def _fuse(uaf, ufnf, aggA2, aggM2, aggF2, cnt2, ab2d, mb2d, p):
    r1 = lambda a: a.reshape(1, -1)
    return pl.pallas_call(
        _fuse_body,
        out_shape=jax.ShapeDtypeStruct((G, 2), _F32),
        scratch_shapes=[pltpu.VMEM((G, 4 * HID), _F32)],
    )(uaf, ufnf, aggA2, aggM2, aggF2, cnt2, ab2d, mb2d,
      r1(p['ac_bias']), r1(p['mc_bias']),
      p['la_Wq'], p['la_Wk'], p['la_Wv'], p['la_Wo'], r1(p['la_bo']),
      p['gm_Wih'], p['gm_Whh'], r1(p['gm_bih']), r1(p['gm_bhh']),
      p['ga_Wih'], p['ga_Whh'], r1(p['ga_bih']), r1(p['ga_bhh']),
      p['cp_W1'], r1(p['cp_b1']), p['cp_W2'], r1(p['cp_b2']),
      p['cm_W1'], r1(p['cm_b1']), p['cm_W2'], r1(p['cm_b2']),
      p['cq_W'], r1(p['cq_b']))


def _pad_idx(idx, n, fill):
    return jnp.concatenate(
        [idx, jnp.full((n - idx.shape[0],), fill, jnp.int32)]).reshape(-1, CHUNK)


def kernel(af, bf, fnf, fef, atom_edge_index, motif_edge_index, a2f_index,
           atom_batch, motif_batch, params):
    p = params
    srcA2d = _pad_idx(atom_edge_index[0], EA_PAD, 0)
    dstA2d = _pad_idx(atom_edge_index[1], EA_PAD, NA_BUF - 1)
    srcM2d = _pad_idx(motif_edge_index[0], EM_PAD, 0)
    dstM2d = _pad_idx(motif_edge_index[1], EM_PAD, NM_BUF - 1)
    a2f2d = _pad_idx(a2f_index, NA_BUF, NM_BUF - 1)
    bf_p = jnp.concatenate([bf, jnp.zeros((EA_PAD - E_ATOM, bf.shape[1]), _F32)])
    fef_p = jnp.concatenate([fef, jnp.zeros((EM_PAD - E_MOTIF, fef.shape[1]), _F32)])

    uaf, ufnf = _encoders(af, fnf, p)
    xgA, xgM = _sc_gather(uaf, ufnf, srcA2d, srcM2d)
    msgA = _edge_messages(bf_p, xgA, p['ac_We'], p['ac_be'], 2048)
    msgM = _edge_messages(fef_p, xgM, p['mc_We'], p['mc_be'], 2048)
    aggA2, aggM2, aggF2, cnt2 = _sc_scatter(msgA, msgM, dstA2d, dstM2d, a2f2d)
    return _fuse(uaf, ufnf, aggA2, aggM2, aggF2, cnt2,
                 atom_batch.reshape(-1, 1), motif_batch.reshape(-1, 1), p)


# speedup vs baseline: 2.1843x; 1.8302x over previous
import functools

import jax
import jax.numpy as jnp
from jax import lax
from jax.experimental import pallas as pl
from jax.experimental.pallas import tpu as pltpu
from jax.experimental.pallas import tpu_sc as plsc

HID = 16
HEADS = 4
G = 64
NC, NS, L = 2, 16, 16
NW = NC * NS
CHUNK = 128

N_ATOM, E_ATOM = 10000, 160000
N_MOTIF, E_MOTIF = 2000, 8000

EA_PAD = 163840
EM_PAD = 8192
NA_BUF = 12288
NM_BUF = 2048

CA = EA_PAD // NW // CHUNK
CM = EM_PAD // NW // CHUNK
CF = NA_BUF // NS // CHUNK

_F32 = jnp.float32


def _lbr(x, W, b):
    h = jnp.dot(x, W, preferred_element_type=_F32, precision=lax.Precision.HIGHEST) + b
    mu = jnp.mean(h, axis=0, keepdims=True)
    var = jnp.mean(h * h, axis=0, keepdims=True) - mu * mu
    h = (h - mu) * lax.rsqrt(var + 1e-5)
    return jnp.maximum(h, 0.0)


def _encoder_body(af, fnf, aW1, ab1, aW2, ab2, mW1, mb1, mW2, mb2,
                  uaf_o, ufnf_o):
    uaf_o[...] = _lbr(_lbr(af[...], aW1[...], ab1[...]), aW2[...], ab2[...])
    ufnf_o[...] = _lbr(_lbr(fnf[...], mW1[...], mb1[...]), mW2[...], mb2[...])


def _encoders(af, fnf, p):
    return pl.pallas_call(
        _encoder_body,
        out_shape=[jax.ShapeDtypeStruct((N_ATOM, HID), _F32),
                   jax.ShapeDtypeStruct((N_MOTIF, HID), _F32)],
    )(af, fnf,
      p['ae_W1'], p['ae_b1'].reshape(1, HID), p['ae_W2'], p['ae_b2'].reshape(1, HID),
      p['me_W1'], p['me_b1'].reshape(1, HID), p['me_W2'], p['me_b2'].reshape(1, HID))


def _gather_body(srcA, srcM, uaf, ufnf, outA, outM,
                 idxA_v, rowsA_v, idxM_v, rowsM_v, sem):
    s = lax.axis_index("s")
    c = lax.axis_index("c")
    wid = s * NC + c

    pltpu.sync_copy(srcA.at[pl.ds(wid * CA, CA)], idxA_v)
    pltpu.sync_copy(srcM.at[pl.ds(wid * CM, CM)], idxM_v)

    def fire8(i, _):
        descs = []
        for b in range(8):
            j = i * 8 + b
            descs.append(pltpu.async_copy(
                uaf.at[idxA_v.at[j]], rowsA_v.at[pl.ds(j * CHUNK, CHUNK)], sem))
        for d in descs:
            d.wait()
        return 0

    lax.fori_loop(0, CA // 8, fire8, 0)

    descs = []
    for j in range(CM):
        descs.append(pltpu.async_copy(
            ufnf.at[idxM_v.at[j]], rowsM_v.at[pl.ds(j * CHUNK, CHUNK)], sem))
    for d in descs:
        d.wait()

    pltpu.sync_copy(rowsA_v, outA.at[pl.ds(wid * CA * CHUNK, CA * CHUNK)])
    pltpu.sync_copy(rowsM_v, outM.at[pl.ds(wid * CM * CHUNK, CM * CHUNK)])


def _sc_gather(uaf, ufnf, srcA2d, srcM2d):
    k = functools.partial(
        pl.kernel,
        out_type=[jax.ShapeDtypeStruct((EA_PAD, HID), _F32),
                  jax.ShapeDtypeStruct((EM_PAD, HID), _F32)],
        mesh=plsc.VectorSubcoreMesh(core_axis_name="c", subcore_axis_name="s"),
        scratch_types=[
            pltpu.VMEM((CA, CHUNK), jnp.int32),
            pltpu.VMEM((CA * CHUNK, HID), _F32),
            pltpu.VMEM((CM, CHUNK), jnp.int32),
            pltpu.VMEM((CM * CHUNK, HID), _F32),
            pltpu.SemaphoreType.DMA,
        ],
        compiler_params=pltpu.CompilerParams(use_tc_tiling_on_sc=False),
    )(_gather_body)
    return k(srcA2d, srcM2d, uaf, ufnf)


def _msg_body(efT, xgT, U2T, outT, blk):
    hp = lax.Precision.HIGHEST
    VT = jnp.dot(U2T[...], xgT[...], preferred_element_type=_F32, precision=hp)
    Q3 = VT[:HID * HID, :].reshape(HID, HID, blk)
    ef3 = efT[...].reshape(HID, 1, blk)
    outT[...] = jnp.sum(Q3 * ef3, axis=0) + VT[HID * HID:, :]


def _edge_messages(efT, xgT, We, be, blk):
    fdim, e_pad = efT.shape
    grid = e_pad // blk
    U = We.reshape(HID, HID, HID).transpose(1, 0, 2).reshape(HID, HID * HID)
    U2T = jnp.concatenate([U, be.reshape(HID, HID)], axis=1).T
    return pl.pallas_call(
        functools.partial(_msg_body, blk=blk),
        grid=(grid,),
        in_specs=[
            pl.BlockSpec((fdim, blk), lambda i: (0, i)),
            pl.BlockSpec((HID, blk), lambda i: (0, i)),
            pl.BlockSpec((HID * HID + HID, HID), lambda i: (0, 0)),
        ],
        out_specs=pl.BlockSpec((HID, blk), lambda i: (0, i)),
        out_shape=jax.ShapeDtypeStruct((HID, e_pad), _F32),
    )(efT, xgT, U2T)


def _scatter_body(msgA, msgM, dstA, dstM, a2f,
                  aggA_o, aggM_o, aggF_o, cnt_o,
                  mbuf, idxA_v, idxM_v, idxF_v, stage_v, ones_v,
                  aggA_sh, aggM_sh, aggF_sh, cnt_sh, sem):
    s = lax.axis_index("s")
    c = lax.axis_index("c")
    wid = s * NC + c
    rows_a = NA_BUF // NS
    rows_m = NM_BUF // NS

    def zrow(i, _):
        stage_v[i] = jnp.zeros((L,), _F32)
        return 0
    lax.fori_loop(0, rows_a, zrow, 0)

    def orow(i, _):
        ones_v[i] = jnp.ones((L,), _F32)
        return 0
    lax.fori_loop(0, CHUNK, orow, 0)

    pltpu.sync_copy(stage_v, aggA_sh.at[pl.ds(s * rows_a, rows_a)])
    pltpu.sync_copy(stage_v.at[pl.ds(0, rows_m)], aggM_sh.at[pl.ds(s * rows_m, rows_m)])
    pltpu.sync_copy(stage_v.at[pl.ds(0, rows_m)], aggF_sh.at[pl.ds(s * rows_m, rows_m)])
    pltpu.sync_copy(stage_v.at[pl.ds(0, rows_m)], cnt_sh.at[pl.ds(s * rows_m, rows_m)])
    plsc.subcore_barrier()

    pltpu.sync_copy(dstA.at[pl.ds(wid * CA, CA)], idxA_v)
    pltpu.sync_copy(msgA.at[pl.ds(wid * CA * CHUNK, CA * CHUNK)], mbuf)

    def fire8(i, _):
        descs = []
        for b in range(8):
            j = i * 8 + b
            descs.append(pltpu.async_copy(
                mbuf.at[pl.ds(j * CHUNK, CHUNK)], aggA_sh.at[idxA_v.at[j]],
                sem, add=True))
        for d in descs:
            d.wait()
        return 0
    lax.fori_loop(0, CA // 8, fire8, 0)

    pltpu.sync_copy(dstM.at[pl.ds(wid * CM, CM)], idxM_v)
    pltpu.sync_copy(msgM.at[pl.ds(wid * CM * CHUNK, CM * CHUNK)],
                    mbuf.at[pl.ds(0, CM * CHUNK)])
    descs = []
    for j in range(CM):
        descs.append(pltpu.async_copy(
            mbuf.at[pl.ds(j * CHUNK, CHUNK)], aggM_sh.at[idxM_v.at[j]],
            sem, add=True))
    for d in descs:
        d.wait()
    plsc.subcore_barrier()

    pltpu.sync_copy(aggA_sh.at[pl.ds(s * rows_a, rows_a)], stage_v)
    pltpu.sync_copy(a2f.at[pl.ds(s * CF, CF)], idxF_v)
    descs = []
    for j in range(CF):
        descs.append(pltpu.async_copy(
            stage_v.at[pl.ds(j * CHUNK, CHUNK)], aggF_sh.at[idxF_v.at[j]],
            sem, add=True))
        descs.append(pltpu.async_copy(
            ones_v, cnt_sh.at[idxF_v.at[j]], sem, add=True))
    for d in descs:
        d.wait()
    plsc.subcore_barrier()

    pltpu.sync_copy(aggA_sh.at[pl.ds(s * rows_a, rows_a)],
                    aggA_o.at[c, pl.ds(s * rows_a, rows_a)])
    pltpu.sync_copy(aggM_sh.at[pl.ds(s * rows_m, rows_m)],
                    aggM_o.at[c, pl.ds(s * rows_m, rows_m)])
    pltpu.sync_copy(aggF_sh.at[pl.ds(s * rows_m, rows_m)],
                    aggF_o.at[c, pl.ds(s * rows_m, rows_m)])
    pltpu.sync_copy(cnt_sh.at[pl.ds(s * rows_m, rows_m)],
                    cnt_o.at[c, pl.ds(s * rows_m, rows_m)])


def _sc_scatter(msgA, msgM, dstA2d, dstM2d, a2f2d):
    k = functools.partial(
        pl.kernel,
        out_type=[jax.ShapeDtypeStruct((NC, NA_BUF, HID), _F32),
                  jax.ShapeDtypeStruct((NC, NM_BUF, HID), _F32),
                  jax.ShapeDtypeStruct((NC, NM_BUF, HID), _F32),
                  jax.ShapeDtypeStruct((NC, NM_BUF, HID), _F32)],
        mesh=plsc.VectorSubcoreMesh(core_axis_name="c", subcore_axis_name="s"),
        scratch_types=[
            pltpu.VMEM((CA * CHUNK, HID), _F32),
            pltpu.VMEM((CA, CHUNK), jnp.int32),
            pltpu.VMEM((CM, CHUNK), jnp.int32),
            pltpu.VMEM((CF, CHUNK), jnp.int32),
            pltpu.VMEM((NA_BUF // NS, HID), _F32),
            pltpu.VMEM((CHUNK, HID), _F32),
            pltpu.VMEM_SHARED((NA_BUF, HID), _F32),
            pltpu.VMEM_SHARED((NM_BUF, HID), _F32),
            pltpu.VMEM_SHARED((NM_BUF, HID), _F32),
            pltpu.VMEM_SHARED((NM_BUF, HID), _F32),
            pltpu.SemaphoreType.DMA,
        ],
        compiler_params=pltpu.CompilerParams(use_tc_tiling_on_sc=False),
    )(_scatter_body)
    return k(msgA, msgM, dstA2d, dstM2d, a2f2d)


def _sigmoid(x):
    return 1.0 / (1.0 + jnp.exp(-x))


def _gru(x, h, Wih, Whh, bih, bhh):
    gi = jnp.dot(x, Wih, preferred_element_type=_F32) + bih
    gh = jnp.dot(h, Whh, preferred_element_type=_F32) + bhh
    r = _sigmoid(gi[:, 0:HID] + gh[:, 0:HID])
    z = _sigmoid(gi[:, HID:2 * HID] + gh[:, HID:2 * HID])
    n = jnp.tanh(gi[:, 2 * HID:] + r * gh[:, 2 * HID:])
    return (1.0 - z) * n + z * h


def _fuse_body(uaf, ufnf, aggA2, aggM2, aggF2, cnt2, ab, mb,
               ac_bias, mc_bias, Wq, Wk, Wv, Wo, bo,
               gmWih, gmWhh, gmbih, gmbhh, gaWih, gaWhh, gabih, gabhh,
               cpW1, cpb1, cpW2, cpb2, cmW1, cmb1, cmW2, cmb2, cqW, cqb,
               out, comb_ref):
    uam = aggA2[0, :N_ATOM, :] + aggA2[1, :N_ATOM, :] + ac_bias[...]
    ufnm = aggM2[0, :N_MOTIF, :] + aggM2[1, :N_MOTIF, :] + mc_bias[...]
    agg_uam = (aggF2[0, :N_MOTIF, :] + aggF2[1, :N_MOTIF, :]
               + cnt2[0, :N_MOTIF, :] * ac_bias[...])

    ufnf_v = ufnf[...]
    uaf_v = uaf[...]

    dk = HID // HEADS
    ii = lax.broadcasted_iota(jnp.int32, (HID, HEADS), 0) // dk
    hh = lax.broadcasted_iota(jnp.int32, (HID, HEADS), 1)
    GH = (ii == hh).astype(_F32)
    GHT = GH.T

    Q = jnp.dot(ufnf_v, Wq[...], preferred_element_type=_F32)
    Kf = jnp.dot(agg_uam, Wk[...], preferred_element_type=_F32)
    Kc = jnp.dot(ufnm, Wk[...], preferred_element_type=_F32)
    Vf = jnp.dot(agg_uam, Wv[...], preferred_element_type=_F32)
    Vc = jnp.dot(ufnm, Wv[...], preferred_element_type=_F32)
    s0 = jnp.dot(Q * Kf, GH, preferred_element_type=_F32) / dk
    s1 = jnp.dot(Q * Kc, GH, preferred_element_type=_F32) / dk
    m = jnp.maximum(s0, s1)
    e0 = jnp.exp(s0 - m)
    e1 = jnp.exp(s1 - m)
    w0 = e0 / (e0 + e1)
    w1 = 1.0 - w0
    att = (jnp.dot(w0, GHT, preferred_element_type=_F32) * Vf
           + jnp.dot(w1, GHT, preferred_element_type=_F32) * Vc)
    motif_msg = jnp.dot(att, Wo[...], preferred_element_type=_F32) + bo[...]

    ufnf_n = _gru(motif_msg, ufnf_v, gmWih[...], gmWhh[...], gmbih[...], gmbhh[...])
    uaf_n = _gru(uam, uaf_v, gaWih[...], gaWhh[...], gabih[...], gabhh[...])

    ab_v = ab[...]
    mb_v = mb[...]
    neg = jnp.float32(-3.0e38)

    def rd(g, _):
        amask = (ab_v == g)
        af32 = amask.astype(_F32)
        acnt = jnp.sum(af32)
        asum = jnp.sum(af32 * uaf_n, axis=0, keepdims=True)
        amean = asum / jnp.maximum(acnt, 1.0)
        amax = jnp.max(jnp.where(amask, uaf_n, neg), axis=0, keepdims=True)
        amax = jnp.where(acnt > 0, amax, 0.0)

        mmask = (mb_v == g)
        mf32 = mmask.astype(_F32)
        mcnt = jnp.sum(mf32)
        msum = jnp.sum(mf32 * ufnf_n, axis=0, keepdims=True)
        mmean = msum / jnp.maximum(mcnt, 1.0)
        mmax = jnp.max(jnp.where(mmask, ufnf_n, neg), axis=0, keepdims=True)
        mmax = jnp.where(mcnt > 0, mmax, 0.0)

        comb_ref[pl.ds(g, 1), :] = jnp.concatenate(
            [amean, amax, mmean, mmax], axis=1)
        return 0

    lax.fori_loop(0, G, rd, 0)

    comb = comb_ref[...]
    rep = jnp.maximum(jnp.dot(comb, cpW1[...], preferred_element_type=_F32)
                      + cpb1[...], 0.0)
    rep = jnp.dot(rep, cpW2[...], preferred_element_type=_F32) + cpb2[...]
    h3 = jnp.maximum(jnp.dot(rep, cmW1[...], preferred_element_type=_F32)
                     + cmb1[...], 0.0)
    lg = jnp.dot(h3, cmW2[...], preferred_element_type=_F32) + cmb2[...]
    out[...] = jnp.dot(lg, cqW[...], preferred_element_type=_F32) + cqb[...]


def _fuse(uaf, ufnf, aggA2, aggM2, aggF2, cnt2, ab2d, mb2d, p):
    r1 = lambda a: a.reshape(1, -1)
    return pl.pallas_call(
        _fuse_body,
        out_shape=jax.ShapeDtypeStruct((G, 2), _F32),
        scratch_shapes=[pltpu.VMEM((G, 4 * HID), _F32)],
    )(uaf, ufnf, aggA2, aggM2, aggF2, cnt2, ab2d, mb2d,
      r1(p['ac_bias']), r1(p['mc_bias']),
      p['la_Wq'], p['la_Wk'], p['la_Wv'], p['la_Wo'], r1(p['la_bo']),
      p['gm_Wih'], p['gm_Whh'], r1(p['gm_bih']), r1(p['gm_bhh']),
      p['ga_Wih'], p['ga_Whh'], r1(p['ga_bih']), r1(p['ga_bhh']),
      p['cp_W1'], r1(p['cp_b1']), p['cp_W2'], r1(p['cp_b2']),
      p['cm_W1'], r1(p['cm_b1']), p['cm_W2'], r1(p['cm_b2']),
      p['cq_W'], r1(p['cq_b']))


def _pad_idx(idx, n, fill):
    return jnp.concatenate(
        [idx, jnp.full((n - idx.shape[0],), fill, jnp.int32)]).reshape(-1, CHUNK)


def kernel(af, bf, fnf, fef, atom_edge_index, motif_edge_index, a2f_index,
           atom_batch, motif_batch, params):
    p = params
    srcA2d = _pad_idx(atom_edge_index[0], EA_PAD, 0)
    dstA2d = _pad_idx(atom_edge_index[1], EA_PAD, NA_BUF - 1)
    srcM2d = _pad_idx(motif_edge_index[0], EM_PAD, 0)
    dstM2d = _pad_idx(motif_edge_index[1], EM_PAD, NM_BUF - 1)
    a2f2d = _pad_idx(a2f_index, NA_BUF, NM_BUF - 1)
    bf_p = jnp.concatenate([bf, jnp.zeros((EA_PAD - E_ATOM, bf.shape[1]), _F32)])
    fef_p = jnp.concatenate([fef, jnp.zeros((EM_PAD - E_MOTIF, fef.shape[1]), _F32)])

    uaf, ufnf = _encoders(af, fnf, p)
    xgA, xgM = _sc_gather(uaf, ufnf, srcA2d, srcM2d)
    msgA = _edge_messages(bf_p.T, xgA.T, p['ac_We'], p['ac_be'], 2048).T
    msgM = _edge_messages(fef_p.T, xgM.T, p['mc_We'], p['mc_be'], 2048).T
    aggA2, aggM2, aggF2, cnt2 = _sc_scatter(msgA, msgM, dstA2d, dstM2d, a2f2d)
    return _fuse(uaf, ufnf, aggA2, aggM2, aggF2, cnt2,
                 atom_batch.reshape(-1, 1), motif_batch.reshape(-1, 1), p)


# speedup vs baseline: 2.9549x; 1.3528x over previous
import functools

import jax
import jax.numpy as jnp
from jax import lax
from jax.experimental import pallas as pl
from jax.experimental.pallas import tpu as pltpu
from jax.experimental.pallas import tpu_sc as plsc

HID = 16
HEADS = 4
G = 64
NC, NS, L = 2, 16, 16
NW = NC * NS
CHUNK = 128

N_ATOM, E_ATOM = 10000, 160000
N_MOTIF, E_MOTIF = 2000, 8000

EA_PAD = 163840
EM_PAD = 8192
NA_BUF = 12288
NM_BUF = 2048

CA = EA_PAD // NW // CHUNK
CM = EM_PAD // NW // CHUNK
CF = NA_BUF // NS // CHUNK

_F32 = jnp.float32


def _lbr(x, W, b):
    h = jnp.dot(x, W, preferred_element_type=_F32, precision=lax.Precision.HIGHEST) + b
    mu = jnp.mean(h, axis=0, keepdims=True)
    var = jnp.mean(h * h, axis=0, keepdims=True) - mu * mu
    h = (h - mu) * lax.rsqrt(var + 1e-5)
    return jnp.maximum(h, 0.0)


def _encoder_body(af, fnf, aW1, ab1, aW2, ab2, mW1, mb1, mW2, mb2,
                  uaf_o, ufnf_o):
    uaf_o[...] = _lbr(_lbr(af[...], aW1[...], ab1[...]), aW2[...], ab2[...])
    ufnf_o[...] = _lbr(_lbr(fnf[...], mW1[...], mb1[...]), mW2[...], mb2[...])


def _encoders(af, fnf, p):
    return pl.pallas_call(
        _encoder_body,
        out_shape=[jax.ShapeDtypeStruct((N_ATOM, HID), _F32),
                   jax.ShapeDtypeStruct((N_MOTIF, HID), _F32)],
    )(af, fnf,
      p['ae_W1'], p['ae_b1'].reshape(1, HID), p['ae_W2'], p['ae_b2'].reshape(1, HID),
      p['me_W1'], p['me_b1'].reshape(1, HID), p['me_W2'], p['me_b2'].reshape(1, HID))


def _gather_body(srcA, srcM, uaf, ufnf, outA, outM,
                 idxA_v, rowsA_v, idxM_v, rowsM_v, sem):
    s = lax.axis_index("s")
    c = lax.axis_index("c")
    wid = s * NC + c

    pltpu.sync_copy(srcA.at[pl.ds(wid * CA, CA)], idxA_v)
    pltpu.sync_copy(srcM.at[pl.ds(wid * CM, CM)], idxM_v)

    def fire8(i, _):
        descs = []
        for b in range(8):
            j = i * 8 + b
            descs.append(pltpu.async_copy(
                uaf.at[idxA_v.at[j]], rowsA_v.at[pl.ds(j * CHUNK, CHUNK)], sem))
        for d in descs:
            d.wait()
        return 0

    lax.fori_loop(0, CA // 8, fire8, 0)

    descs = []
    for j in range(CM):
        descs.append(pltpu.async_copy(
            ufnf.at[idxM_v.at[j]], rowsM_v.at[pl.ds(j * CHUNK, CHUNK)], sem))
    for d in descs:
        d.wait()

    pltpu.sync_copy(rowsA_v, outA.at[pl.ds(wid * CA * CHUNK, CA * CHUNK)])
    pltpu.sync_copy(rowsM_v, outM.at[pl.ds(wid * CM * CHUNK, CM * CHUNK)])


def _sc_gather(uaf, ufnf, srcA2d, srcM2d):
    k = functools.partial(
        pl.kernel,
        out_type=[jax.ShapeDtypeStruct((EA_PAD, HID), _F32),
                  jax.ShapeDtypeStruct((EM_PAD, HID), _F32)],
        mesh=plsc.VectorSubcoreMesh(core_axis_name="c", subcore_axis_name="s"),
        scratch_types=[
            pltpu.VMEM((CA, CHUNK), jnp.int32),
            pltpu.VMEM((CA * CHUNK, HID), _F32),
            pltpu.VMEM((CM, CHUNK), jnp.int32),
            pltpu.VMEM((CM * CHUNK, HID), _F32),
            pltpu.SemaphoreType.DMA,
        ],
        compiler_params=pltpu.CompilerParams(use_tc_tiling_on_sc=False),
    )(_gather_body)
    return k(srcA2d, srcM2d, uaf, ufnf)


def _msg_body(efT, xgT, U2T, outT, blk):
    hp = lax.Precision.HIGHEST
    VT = jnp.dot(U2T[...], xgT[...], preferred_element_type=_F32, precision=hp)
    Q3 = VT[:HID * HID, :].reshape(HID, HID, blk)
    ef3 = efT[...].reshape(HID, 1, blk)
    outT[...] = jnp.sum(Q3 * ef3, axis=0) + VT[HID * HID:, :]


def _edge_messages(efT, xgT, We, be, blk):
    fdim, e_pad = efT.shape
    grid = e_pad // blk
    U = We.reshape(HID, HID, HID).transpose(1, 0, 2).reshape(HID, HID * HID)
    U2T = jnp.concatenate([U, be.reshape(HID, HID)], axis=1).T
    return pl.pallas_call(
        functools.partial(_msg_body, blk=blk),
        grid=(grid,),
        in_specs=[
            pl.BlockSpec((fdim, blk), lambda i: (0, i)),
            pl.BlockSpec((HID, blk), lambda i: (0, i)),
            pl.BlockSpec((HID * HID + HID, HID), lambda i: (0, 0)),
        ],
        out_specs=pl.BlockSpec((HID, blk), lambda i: (0, i)),
        out_shape=jax.ShapeDtypeStruct((HID, e_pad), _F32),
    )(efT, xgT, U2T)


def _scatter_body(msgA, msgM, dstA, dstM, a2f,
                  aggA_o, aggM_o, aggF_o, cnt_o,
                  mbuf, idxA_v, idxM_v, idxF_v, stage_v, ones_v,
                  aggA_sh, aggM_sh, aggF_sh, cnt_sh, sem):
    s = lax.axis_index("s")
    c = lax.axis_index("c")
    wid = s * NC + c
    rows_a = NA_BUF // NS
    rows_m = NM_BUF // NS

    def zrow(i, _):
        stage_v[i] = jnp.zeros((L,), _F32)
        return 0
    lax.fori_loop(0, rows_a, zrow, 0)

    def orow(i, _):
        ones_v[i] = jnp.ones((L,), _F32)
        return 0
    lax.fori_loop(0, CHUNK, orow, 0)

    pltpu.sync_copy(stage_v, aggA_sh.at[pl.ds(s * rows_a, rows_a)])
    pltpu.sync_copy(stage_v.at[pl.ds(0, rows_m)], aggM_sh.at[pl.ds(s * rows_m, rows_m)])
    pltpu.sync_copy(stage_v.at[pl.ds(0, rows_m)], aggF_sh.at[pl.ds(s * rows_m, rows_m)])
    pltpu.sync_copy(stage_v.at[pl.ds(0, rows_m)], cnt_sh.at[pl.ds(s * rows_m, rows_m)])
    plsc.subcore_barrier()

    pltpu.sync_copy(dstA.at[pl.ds(wid * CA, CA)], idxA_v)
    pltpu.sync_copy(msgA.at[pl.ds(wid * CA * CHUNK, CA * CHUNK)], mbuf)

    def fire8(i, _):
        descs = []
        for b in range(8):
            j = i * 8 + b
            descs.append(pltpu.async_copy(
                mbuf.at[pl.ds(j * CHUNK, CHUNK)], aggA_sh.at[idxA_v.at[j]],
                sem, add=True))
        for d in descs:
            d.wait()
        return 0
    lax.fori_loop(0, CA // 8, fire8, 0)

    pltpu.sync_copy(dstM.at[pl.ds(wid * CM, CM)], idxM_v)
    pltpu.sync_copy(msgM.at[pl.ds(wid * CM * CHUNK, CM * CHUNK)],
                    mbuf.at[pl.ds(0, CM * CHUNK)])
    descs = []
    for j in range(CM):
        descs.append(pltpu.async_copy(
            mbuf.at[pl.ds(j * CHUNK, CHUNK)], aggM_sh.at[idxM_v.at[j]],
            sem, add=True))
    for d in descs:
        d.wait()
    plsc.subcore_barrier()

    pltpu.sync_copy(aggA_sh.at[pl.ds(s * rows_a, rows_a)], stage_v)
    pltpu.sync_copy(a2f.at[pl.ds(s * CF, CF)], idxF_v)
    descs = []
    for j in range(CF):
        descs.append(pltpu.async_copy(
            stage_v.at[pl.ds(j * CHUNK, CHUNK)], aggF_sh.at[idxF_v.at[j]],
            sem, add=True))
        descs.append(pltpu.async_copy(
            ones_v, cnt_sh.at[idxF_v.at[j]], sem, add=True))
    for d in descs:
        d.wait()
    plsc.subcore_barrier()

    pltpu.sync_copy(aggA_sh.at[pl.ds(s * rows_a, rows_a)],
                    aggA_o.at[c, pl.ds(s * rows_a, rows_a)])
    pltpu.sync_copy(aggM_sh.at[pl.ds(s * rows_m, rows_m)],
                    aggM_o.at[c, pl.ds(s * rows_m, rows_m)])
    pltpu.sync_copy(aggF_sh.at[pl.ds(s * rows_m, rows_m)],
                    aggF_o.at[c, pl.ds(s * rows_m, rows_m)])
    pltpu.sync_copy(cnt_sh.at[pl.ds(s * rows_m, rows_m)],
                    cnt_o.at[c, pl.ds(s * rows_m, rows_m)])


def _sc_scatter(msgA, msgM, dstA2d, dstM2d, a2f2d):
    k = functools.partial(
        pl.kernel,
        out_type=[jax.ShapeDtypeStruct((NC, NA_BUF, HID), _F32),
                  jax.ShapeDtypeStruct((NC, NM_BUF, HID), _F32),
                  jax.ShapeDtypeStruct((NC, NM_BUF, HID), _F32),
                  jax.ShapeDtypeStruct((NC, NM_BUF, HID), _F32)],
        mesh=plsc.VectorSubcoreMesh(core_axis_name="c", subcore_axis_name="s"),
        scratch_types=[
            pltpu.VMEM((CA * CHUNK, HID), _F32),
            pltpu.VMEM((CA, CHUNK), jnp.int32),
            pltpu.VMEM((CM, CHUNK), jnp.int32),
            pltpu.VMEM((CF, CHUNK), jnp.int32),
            pltpu.VMEM((NA_BUF // NS, HID), _F32),
            pltpu.VMEM((CHUNK, HID), _F32),
            pltpu.VMEM_SHARED((NA_BUF, HID), _F32),
            pltpu.VMEM_SHARED((NM_BUF, HID), _F32),
            pltpu.VMEM_SHARED((NM_BUF, HID), _F32),
            pltpu.VMEM_SHARED((NM_BUF, HID), _F32),
            pltpu.SemaphoreType.DMA,
        ],
        compiler_params=pltpu.CompilerParams(use_tc_tiling_on_sc=False),
    )(_scatter_body)
    return k(msgA, msgM, dstA2d, dstM2d, a2f2d)


NA_RD = 10240
NM_RD = 2048
GBUF = 80
RA = NA_RD // NW
RM = NM_RD // NW
RG = GBUF // NS


def _sigmoid(x):
    return 1.0 / (1.0 + jnp.exp(-x))


def _gru(x, h, Wih, Whh, bih, bhh):
    gi = jnp.dot(x, Wih, preferred_element_type=_F32) + bih
    gh = jnp.dot(h, Whh, preferred_element_type=_F32) + bhh
    r = _sigmoid(gi[:, 0:HID] + gh[:, 0:HID])
    z = _sigmoid(gi[:, HID:2 * HID] + gh[:, HID:2 * HID])
    n = jnp.tanh(gi[:, 2 * HID:] + r * gh[:, 2 * HID:])
    return (1.0 - z) * n + z * h


def _e1_body(uaf, ufnf, aggA2, aggM2, aggF2, cnt2,
             ac_bias, mc_bias, Wq, Wk, Wv, Wo, bo,
             gmWih, gmWhh, gmbih, gmbhh, gaWih, gaWhh, gabih, gabhh,
             uafn_o, ufnfn_o):
    uam = aggA2[0, :N_ATOM, :] + aggA2[1, :N_ATOM, :] + ac_bias[...]
    ufnm = aggM2[0, :N_MOTIF, :] + aggM2[1, :N_MOTIF, :] + mc_bias[...]
    agg_uam = (aggF2[0, :N_MOTIF, :] + aggF2[1, :N_MOTIF, :]
               + cnt2[0, :N_MOTIF, :] * ac_bias[...])

    ufnf_v = ufnf[...]
    uaf_v = uaf[...]

    dk = HID // HEADS
    ii = lax.broadcasted_iota(jnp.int32, (HID, HEADS), 0) // dk
    hh = lax.broadcasted_iota(jnp.int32, (HID, HEADS), 1)
    GH = (ii == hh).astype(_F32)
    GHT = GH.T

    Q = jnp.dot(ufnf_v, Wq[...], preferred_element_type=_F32)
    Kf = jnp.dot(agg_uam, Wk[...], preferred_element_type=_F32)
    Kc = jnp.dot(ufnm, Wk[...], preferred_element_type=_F32)
    Vf = jnp.dot(agg_uam, Wv[...], preferred_element_type=_F32)
    Vc = jnp.dot(ufnm, Wv[...], preferred_element_type=_F32)
    s0 = jnp.dot(Q * Kf, GH, preferred_element_type=_F32) / dk
    s1 = jnp.dot(Q * Kc, GH, preferred_element_type=_F32) / dk
    m = jnp.maximum(s0, s1)
    e0 = jnp.exp(s0 - m)
    e1 = jnp.exp(s1 - m)
    w0 = e0 / (e0 + e1)
    w1 = 1.0 - w0
    att = (jnp.dot(w0, GHT, preferred_element_type=_F32) * Vf
           + jnp.dot(w1, GHT, preferred_element_type=_F32) * Vc)
    motif_msg = jnp.dot(att, Wo[...], preferred_element_type=_F32) + bo[...]

    ufnfn_o[0:N_MOTIF, :] = _gru(motif_msg, ufnf_v, gmWih[...], gmWhh[...],
                                 gmbih[...], gmbhh[...])
    ufnfn_o[N_MOTIF:, :] = jnp.zeros((NM_RD - N_MOTIF, HID), _F32)
    uafn_o[0:N_ATOM, :] = _gru(uam, uaf_v, gaWih[...], gaWhh[...],
                               gabih[...], gabhh[...])
    uafn_o[N_ATOM:, :] = jnp.zeros((NA_RD - N_ATOM, HID), _F32)


def _e1(uaf, ufnf, aggA2, aggM2, aggF2, cnt2, p):
    r1 = lambda a: a.reshape(1, -1)
    return pl.pallas_call(
        _e1_body,
        out_shape=[jax.ShapeDtypeStruct((NA_RD, HID), _F32),
                   jax.ShapeDtypeStruct((NM_RD, HID), _F32)],
    )(uaf, ufnf, aggA2, aggM2, aggF2, cnt2,
      r1(p['ac_bias']), r1(p['mc_bias']),
      p['la_Wq'], p['la_Wk'], p['la_Wv'], p['la_Wo'], r1(p['la_bo']),
      p['gm_Wih'], p['gm_Whh'], r1(p['gm_bih']), r1(p['gm_bhh']),
      p['ga_Wih'], p['ga_Whh'], r1(p['ga_bih']), r1(p['ga_bhh']))


def _readout_body(nfa, nfm, ab, mb,
                  asum_o, amax_o, acnt_o, msum_o, mmax_o, mcnt_o,
                  rows_v, idx_v, bsum_a, bmax_a, bcnt_a, bsum_m, bmax_m, bcnt_m,
                  slab_v, res_v, sh_list0, sh_list1, sh_list2, sh_list3,
                  sh_list4, sh_list5, sem):
    s = lax.axis_index("s")
    c = lax.axis_index("c")
    wid = s * NC + c
    neg = jnp.full((L,), -3.0e38, _F32)
    zero = jnp.zeros((L,), _F32)

    def init(i, _):
        bsum_a[i] = zero
        bmax_a[i] = neg
        bcnt_a[i] = zero
        bsum_m[i] = zero
        bmax_m[i] = neg
        bcnt_m[i] = zero
        return 0
    lax.fori_loop(0, GBUF, init, 0)

    one = jnp.ones((L,), _F32)

    pltpu.sync_copy(nfa.at[pl.ds(wid * RA, RA)], rows_v)
    pltpu.sync_copy(ab.at[pl.ds(wid * RA, RA)], idx_v)

    def arow(rb, _):
        r0 = rb * L
        gvec = idx_v[pl.ds(r0, L)]
        for j in range(L):
            g = gvec[j]
            row = rows_v[r0 + j]
            bsum_a[g] = bsum_a[g] + row
            bmax_a[g] = jnp.maximum(bmax_a[g], row)
            bcnt_a[g] = bcnt_a[g] + one
        return 0
    lax.fori_loop(0, RA // L, arow, 0)

    pltpu.sync_copy(nfm.at[pl.ds(wid * RM, RM)], rows_v.at[pl.ds(0, RM)])
    pltpu.sync_copy(mb.at[pl.ds(wid * RM, RM)], idx_v.at[pl.ds(0, RM)])

    def mrow(rb, _):
        r0 = rb * L
        gvec = idx_v[pl.ds(r0, L)]
        for j in range(L):
            g = gvec[j]
            row = rows_v[r0 + j]
            bsum_m[g] = bsum_m[g] + row
            bmax_m[g] = jnp.maximum(bmax_m[g], row)
            bcnt_m[g] = bcnt_m[g] + one
        return 0
    lax.fori_loop(0, RM // L, mrow, 0)

    shs = [sh_list0, sh_list1, sh_list2, sh_list3, sh_list4, sh_list5]
    bufs = [bsum_a, bmax_a, bcnt_a, bsum_m, bmax_m, bcnt_m]
    outs = [asum_o, amax_o, acnt_o, msum_o, mmax_o, mcnt_o]
    for sh, buf in zip(shs, bufs):
        pltpu.sync_copy(buf, sh.at[s])
    plsc.subcore_barrier()

    for bi, (sh, out, is_max) in enumerate(
            zip(shs, outs, [False, True, False, False, True, False])):
        descs = [pltpu.async_copy(sh.at[t, pl.ds(s * RG, RG)],
                                  slab_v.at[t], sem) for t in range(NS)]
        for d in descs:
            d.wait()

        def comb(r, _):
            acc = slab_v[0, r]
            for t in range(1, NS):
                if is_max:
                    acc = jnp.maximum(acc, slab_v[t, r])
                else:
                    acc = acc + slab_v[t, r]
            res_v[r] = acc
            return 0
        lax.fori_loop(0, RG, comb, 0)
        pltpu.sync_copy(res_v, out.at[c, pl.ds(s * RG, RG)])
        plsc.subcore_barrier()


def _sc_readout(nfa, nfm, ab_p, mb_p):
    out_t = jax.ShapeDtypeStruct((NC, GBUF, HID), _F32)
    k = functools.partial(
        pl.kernel,
        out_type=[out_t] * 6,
        mesh=plsc.VectorSubcoreMesh(core_axis_name="c", subcore_axis_name="s"),
        scratch_types=(
            [pltpu.VMEM((RA, HID), _F32), pltpu.VMEM((RA,), jnp.int32)]
            + [pltpu.VMEM((GBUF, HID), _F32)] * 6
            + [pltpu.VMEM((NS, RG, HID), _F32), pltpu.VMEM((RG, HID), _F32)]
            + [pltpu.VMEM_SHARED((NS, GBUF, HID), _F32)] * 6
            + [pltpu.SemaphoreType.DMA]),
        compiler_params=pltpu.CompilerParams(use_tc_tiling_on_sc=False),
    )(_readout_body)
    return k(nfa, nfm, ab_p, mb_p)


def _e2_body(asum2, amax2, acnt2, msum2, mmax2, mcnt2,
             cpW1, cpb1, cpW2, cpb2, cmW1, cmb1, cmW2, cmb2, cqW, cqb, out):
    asum = asum2[0, :G, :] + asum2[1, :G, :]
    amax = jnp.maximum(amax2[0, :G, :], amax2[1, :G, :])
    acnt = acnt2[0, :G, :] + acnt2[1, :G, :]
    msum = msum2[0, :G, :] + msum2[1, :G, :]
    mmax = jnp.maximum(mmax2[0, :G, :], mmax2[1, :G, :])
    mcnt = mcnt2[0, :G, :] + mcnt2[1, :G, :]
    amean = asum / jnp.maximum(acnt, 1.0)
    amaxf = jnp.where(acnt > 0, amax, 0.0)
    mmean = msum / jnp.maximum(mcnt, 1.0)
    mmaxf = jnp.where(mcnt > 0, mmax, 0.0)
    comb = jnp.concatenate([amean, amaxf, mmean, mmaxf], axis=1)
    rep = jnp.maximum(jnp.dot(comb, cpW1[...], preferred_element_type=_F32)
                      + cpb1[...], 0.0)
    rep = jnp.dot(rep, cpW2[...], preferred_element_type=_F32) + cpb2[...]
    h3 = jnp.maximum(jnp.dot(rep, cmW1[...], preferred_element_type=_F32)
                     + cmb1[...], 0.0)
    lg = jnp.dot(h3, cmW2[...], preferred_element_type=_F32) + cmb2[...]
    out[...] = jnp.dot(lg, cqW[...], preferred_element_type=_F32) + cqb[...]


def _e2(rd6, p):
    r1 = lambda a: a.reshape(1, -1)
    return pl.pallas_call(
        _e2_body,
        out_shape=jax.ShapeDtypeStruct((G, 2), _F32),
    )(*rd6,
      p['cp_W1'], r1(p['cp_b1']), p['cp_W2'], r1(p['cp_b2']),
      p['cm_W1'], r1(p['cm_b1']), p['cm_W2'], r1(p['cm_b2']),
      p['cq_W'], r1(p['cq_b']))


def _pad_idx(idx, n, fill):
    return jnp.concatenate(
        [idx, jnp.full((n - idx.shape[0],), fill, jnp.int32)]).reshape(-1, CHUNK)


def kernel(af, bf, fnf, fef, atom_edge_index, motif_edge_index, a2f_index,
           atom_batch, motif_batch, params):
    p = params
    srcA2d = _pad_idx(atom_edge_index[0], EA_PAD, 0)
    dstA2d = _pad_idx(atom_edge_index[1], EA_PAD, NA_BUF - 1)
    srcM2d = _pad_idx(motif_edge_index[0], EM_PAD, 0)
    dstM2d = _pad_idx(motif_edge_index[1], EM_PAD, NM_BUF - 1)
    a2f2d = _pad_idx(a2f_index, NA_BUF, NM_BUF - 1)
    bf_p = jnp.concatenate([bf, jnp.zeros((EA_PAD - E_ATOM, bf.shape[1]), _F32)])
    fef_p = jnp.concatenate([fef, jnp.zeros((EM_PAD - E_MOTIF, fef.shape[1]), _F32)])

    uaf, ufnf = _encoders(af, fnf, p)
    xgA, xgM = _sc_gather(uaf, ufnf, srcA2d, srcM2d)
    msgA = _edge_messages(bf_p.T, xgA.T, p['ac_We'], p['ac_be'], 2048).T
    msgM = _edge_messages(fef_p.T, xgM.T, p['mc_We'], p['mc_be'], 2048).T
    aggA2, aggM2, aggF2, cnt2 = _sc_scatter(msgA, msgM, dstA2d, dstM2d, a2f2d)
    uafn, ufnfn = _e1(uaf, ufnf, aggA2, aggM2, aggF2, cnt2, p)
    ab_p = jnp.concatenate(
        [atom_batch, jnp.full((NA_RD - N_ATOM,), G, jnp.int32)])
    mb_p = jnp.concatenate(
        [motif_batch, jnp.full((NM_RD - N_MOTIF,), G, jnp.int32)])
    rd6 = _sc_readout(uafn, ufnfn, ab_p, mb_p)
    return _e2(rd6, p)


# speedup vs baseline: 3.2532x; 1.1010x over previous
import functools

import jax
import jax.numpy as jnp
from jax import lax
from jax.experimental import pallas as pl
from jax.experimental.pallas import tpu as pltpu
from jax.experimental.pallas import tpu_sc as plsc

HID = 16
HEADS = 4
G = 64
NC, NS, L = 2, 16, 16
NW = NC * NS
CHUNK = 128

N_ATOM, E_ATOM = 10000, 160000
N_MOTIF, E_MOTIF = 2000, 8000

EA_PAD = 163840
EM_PAD = 8192
NA_BUF = 12288
NM_BUF = 2048

CA = EA_PAD // NW // CHUNK
CM = EM_PAD // NW // CHUNK
CF = NA_BUF // NS // CHUNK

_F32 = jnp.float32

NA_RD = 10240
NM_RD = 2048
GBUF = 80
RA = NA_RD // NW
RM = NM_RD // NW
RG = GBUF // NS


def _lbr(x, W, b):
    h = jnp.dot(x, W, preferred_element_type=_F32, precision=lax.Precision.HIGHEST) + b
    mu = jnp.mean(h, axis=0, keepdims=True)
    var = jnp.mean(h * h, axis=0, keepdims=True) - mu * mu
    h = (h - mu) * lax.rsqrt(var + 1e-5)
    return jnp.maximum(h, 0.0)


def _encoder_body(af, fnf, aW1, ab1, aW2, ab2, mW1, mb1, mW2, mb2,
                  uaf_o, ufnf_o):
    uaf_o[0:N_ATOM, :] = _lbr(_lbr(af[...], aW1[...], ab1[...]),
                              aW2[...], ab2[...])
    uaf_o[N_ATOM:, :] = jnp.zeros((NA_RD - N_ATOM, HID), _F32)
    ufnf_o[0:N_MOTIF, :] = _lbr(_lbr(fnf[...], mW1[...], mb1[...]),
                                mW2[...], mb2[...])
    ufnf_o[N_MOTIF:, :] = jnp.zeros((NM_RD - N_MOTIF, HID), _F32)


def _encoders(af, fnf, p):
    return pl.pallas_call(
        _encoder_body,
        out_shape=[jax.ShapeDtypeStruct((NA_RD, HID), _F32),
                   jax.ShapeDtypeStruct((NM_RD, HID), _F32)],
    )(af, fnf,
      p['ae_W1'], p['ae_b1'].reshape(1, HID), p['ae_W2'], p['ae_b2'].reshape(1, HID),
      p['me_W1'], p['me_b1'].reshape(1, HID), p['me_W2'], p['me_b2'].reshape(1, HID))


def _gather_body(srcA, srcM, uaf, ufnf, outA, outM,
                 idxA_v, rowsA_v, idxM_v, rowsM_v, sem):
    s = lax.axis_index("s")
    c = lax.axis_index("c")
    wid = s * NC + c

    pltpu.sync_copy(srcA.at[pl.ds(wid * CA, CA)], idxA_v)
    pltpu.sync_copy(srcM.at[pl.ds(wid * CM, CM)], idxM_v)

    DEP = 8

    def fire(j):
        return pltpu.async_copy(
            uaf.at[idxA_v.at[j]], rowsA_v.at[pl.ds(j * CHUNK, CHUNK)], sem)

    def wait_one():
        pltpu.make_async_copy(
            uaf.at[idxA_v.at[0]], rowsA_v.at[pl.ds(0, CHUNK)], sem).wait()

    for j in range(DEP):
        fire(j)

    def step(i, _):
        wait_one()
        fire(i + DEP)
        return 0

    lax.fori_loop(0, CA - DEP, step, 0)
    for _ in range(DEP):
        wait_one()

    descs = []
    for j in range(CM):
        descs.append(pltpu.async_copy(
            ufnf.at[idxM_v.at[j]], rowsM_v.at[pl.ds(j * CHUNK, CHUNK)], sem))
    for d in descs:
        d.wait()

    pltpu.sync_copy(rowsA_v, outA.at[pl.ds(wid * CA * CHUNK, CA * CHUNK)])
    pltpu.sync_copy(rowsM_v, outM.at[pl.ds(wid * CM * CHUNK, CM * CHUNK)])


def _sc_gather(uaf, ufnf, srcA2d, srcM2d):
    k = functools.partial(
        pl.kernel,
        out_type=[jax.ShapeDtypeStruct((EA_PAD, HID), _F32),
                  jax.ShapeDtypeStruct((EM_PAD, HID), _F32)],
        mesh=plsc.VectorSubcoreMesh(core_axis_name="c", subcore_axis_name="s"),
        scratch_types=[
            pltpu.VMEM((CA, CHUNK), jnp.int32),
            pltpu.VMEM((CA * CHUNK, HID), _F32),
            pltpu.VMEM((CM, CHUNK), jnp.int32),
            pltpu.VMEM((CM * CHUNK, HID), _F32),
            pltpu.SemaphoreType.DMA,
        ],
        compiler_params=pltpu.CompilerParams(use_tc_tiling_on_sc=False),
    )(_gather_body)
    return k(srcA2d, srcM2d, uaf, ufnf)


def _msg_body(efT, xgT, U2T, outT, blk):
    hp = lax.Precision.HIGHEST
    VT = jnp.dot(U2T[...], xgT[...], preferred_element_type=_F32, precision=hp)
    Q3 = VT[:HID * HID, :].reshape(HID, HID, blk)
    ef3 = efT[...].reshape(HID, 1, blk)
    outT[...] = jnp.sum(Q3 * ef3, axis=0) + VT[HID * HID:, :]


def _edge_messages(efT, xgT, We, be, blk):
    fdim, e_pad = efT.shape
    grid = e_pad // blk
    U = We.reshape(HID, HID, HID).transpose(1, 0, 2).reshape(HID, HID * HID)
    U2T = jnp.concatenate([U, be.reshape(HID, HID)], axis=1).T
    return pl.pallas_call(
        functools.partial(_msg_body, blk=blk),
        grid=(grid,),
        in_specs=[
            pl.BlockSpec((fdim, blk), lambda i: (0, i)),
            pl.BlockSpec((HID, blk), lambda i: (0, i)),
            pl.BlockSpec((HID * HID + HID, HID), lambda i: (0, 0)),
        ],
        out_specs=pl.BlockSpec((HID, blk), lambda i: (0, i)),
        out_shape=jax.ShapeDtypeStruct((HID, e_pad), _F32),
    )(efT, xgT, U2T)


def _scatter_body(msgA, msgM, dstA, dstM, a2f,
                  aggA_o, aggM_o, aggF_o, cnt_o,
                  mbuf, idxA_v, idxM_v, idxF_v, stage_v, ones_v,
                  aggA_sh, aggM_sh, aggF_sh, cnt_sh, sem, semS):
    s = lax.axis_index("s")
    c = lax.axis_index("c")
    wid = s * NC + c
    rows_a = NA_BUF // NS
    rows_m = NM_BUF // NS

    def zrow(i, _):
        stage_v[i] = jnp.zeros((L,), _F32)
        return 0
    lax.fori_loop(0, rows_a, zrow, 0)

    def orow(i, _):
        ones_v[i] = jnp.ones((L,), _F32)
        return 0
    lax.fori_loop(0, CHUNK, orow, 0)

    pltpu.sync_copy(stage_v, aggA_sh.at[pl.ds(s * rows_a, rows_a)])
    pltpu.sync_copy(stage_v.at[pl.ds(0, rows_m)], aggM_sh.at[pl.ds(s * rows_m, rows_m)])
    pltpu.sync_copy(stage_v.at[pl.ds(0, rows_m)], aggF_sh.at[pl.ds(s * rows_m, rows_m)])
    pltpu.sync_copy(stage_v.at[pl.ds(0, rows_m)], cnt_sh.at[pl.ds(s * rows_m, rows_m)])
    plsc.subcore_barrier()

    pltpu.sync_copy(dstA.at[pl.ds(wid * CA, CA)], idxA_v)
    DEP = 8
    base = wid * CA * CHUNK

    def stage(j):
        pltpu.async_copy(msgA.at[pl.ds(base + j * CHUNK, CHUNK)],
                         mbuf.at[pl.ds(j * CHUNK, CHUNK)], semS)

    def waitS():
        pltpu.make_async_copy(msgA.at[pl.ds(base, CHUNK)],
                              mbuf.at[pl.ds(0, CHUNK)], semS).wait()

    def scat(j):
        pltpu.async_copy(mbuf.at[pl.ds(j * CHUNK, CHUNK)],
                         aggA_sh.at[idxA_v.at[j]], sem, add=True)

    def waitC():
        pltpu.make_async_copy(mbuf.at[pl.ds(0, CHUNK)],
                              aggA_sh.at[idxA_v.at[0]], sem).wait()

    for j in range(DEP):
        stage(j)

    def mstep(i, _):
        waitS()
        scat(i)
        stage(i + DEP)
        return 0
    lax.fori_loop(0, CA - DEP, mstep, 0)

    def tstep(i, _):
        waitS()
        scat(i)
        return 0
    lax.fori_loop(CA - DEP, CA, tstep, 0)

    def dstep(i, _):
        waitC()
        return 0
    lax.fori_loop(0, CA, dstep, 0)

    pltpu.sync_copy(dstM.at[pl.ds(wid * CM, CM)], idxM_v)
    pltpu.sync_copy(msgM.at[pl.ds(wid * CM * CHUNK, CM * CHUNK)],
                    mbuf.at[pl.ds(0, CM * CHUNK)])
    descs = []
    for j in range(CM):
        descs.append(pltpu.async_copy(
            mbuf.at[pl.ds(j * CHUNK, CHUNK)], aggM_sh.at[idxM_v.at[j]],
            sem, add=True))
    for d in descs:
        d.wait()
    plsc.subcore_barrier()

    pltpu.sync_copy(aggA_sh.at[pl.ds(s * rows_a, rows_a)], stage_v)
    pltpu.sync_copy(a2f.at[pl.ds(s * CF, CF)], idxF_v)
    descs = []
    for j in range(CF):
        descs.append(pltpu.async_copy(
            stage_v.at[pl.ds(j * CHUNK, CHUNK)], aggF_sh.at[idxF_v.at[j]],
            sem, add=True))
        descs.append(pltpu.async_copy(
            ones_v, cnt_sh.at[idxF_v.at[j]], sem, add=True))
    for d in descs:
        d.wait()
    plsc.subcore_barrier()

    pltpu.sync_copy(aggA_sh.at[pl.ds(s * rows_a, rows_a)],
                    aggA_o.at[c, pl.ds(s * rows_a, rows_a)])
    pltpu.sync_copy(aggM_sh.at[pl.ds(s * rows_m, rows_m)],
                    aggM_o.at[c, pl.ds(s * rows_m, rows_m)])
    pltpu.sync_copy(aggF_sh.at[pl.ds(s * rows_m, rows_m)],
                    aggF_o.at[c, pl.ds(s * rows_m, rows_m)])
    pltpu.sync_copy(cnt_sh.at[pl.ds(s * rows_m, rows_m)],
                    cnt_o.at[c, pl.ds(s * rows_m, rows_m)])


def _sc_scatter(msgA, msgM, dstA2d, dstM2d, a2f2d):
    k = functools.partial(
        pl.kernel,
        out_type=[jax.ShapeDtypeStruct((NC, NA_BUF, HID), _F32),
                  jax.ShapeDtypeStruct((NC, NM_BUF, HID), _F32),
                  jax.ShapeDtypeStruct((NC, NM_BUF, HID), _F32),
                  jax.ShapeDtypeStruct((NC, NM_BUF, HID), _F32)],
        mesh=plsc.VectorSubcoreMesh(core_axis_name="c", subcore_axis_name="s"),
        scratch_types=[
            pltpu.VMEM((CA * CHUNK, HID), _F32),
            pltpu.VMEM((CA, CHUNK), jnp.int32),
            pltpu.VMEM((CM, CHUNK), jnp.int32),
            pltpu.VMEM((CF, CHUNK), jnp.int32),
            pltpu.VMEM((NA_BUF // NS, HID), _F32),
            pltpu.VMEM((CHUNK, HID), _F32),
            pltpu.VMEM_SHARED((NA_BUF, HID), _F32),
            pltpu.VMEM_SHARED((NM_BUF, HID), _F32),
            pltpu.VMEM_SHARED((NM_BUF, HID), _F32),
            pltpu.VMEM_SHARED((NM_BUF, HID), _F32),
            pltpu.SemaphoreType.DMA,
            pltpu.SemaphoreType.DMA,
        ],
        compiler_params=pltpu.CompilerParams(use_tc_tiling_on_sc=False),
    )(_scatter_body)
    return k(msgA, msgM, dstA2d, dstM2d, a2f2d)


def _sigmoid(x):
    return 1.0 / (1.0 + jnp.exp(-x))


def _gru(x, h, Wih, Whh, bih, bhh):
    gi = jnp.dot(x, Wih, preferred_element_type=_F32, precision=lax.Precision.HIGHEST) + bih
    gh = jnp.dot(h, Whh, preferred_element_type=_F32, precision=lax.Precision.HIGHEST) + bhh
    r = _sigmoid(gi[:, 0:HID] + gh[:, 0:HID])
    z = _sigmoid(gi[:, HID:2 * HID] + gh[:, HID:2 * HID])
    n = jnp.tanh(gi[:, 2 * HID:] + r * gh[:, 2 * HID:])
    return (1.0 - z) * n + z * h


def _e1m_body(ufnf, aggM2, aggF2, cnt2,
              ac_bias, mc_bias, Wq, Wk, Wv, Wo, bo,
              gmWih, gmWhh, gmbih, gmbhh, ufnfn_o):
    hp = lax.Precision.HIGHEST
    ufnm = aggM2[0, :N_MOTIF, :] + aggM2[1, :N_MOTIF, :] + mc_bias[...]
    agg_uam = (aggF2[0, :N_MOTIF, :] + aggF2[1, :N_MOTIF, :]
               + cnt2[0, :N_MOTIF, :] * ac_bias[...])
    ufnf_v = ufnf[0:N_MOTIF, :]

    dk = HID // HEADS
    ii = lax.broadcasted_iota(jnp.int32, (HID, HEADS), 0) // dk
    hh = lax.broadcasted_iota(jnp.int32, (HID, HEADS), 1)
    GH = (ii == hh).astype(_F32)
    GHT = GH.T

    Q = jnp.dot(ufnf_v, Wq[...], preferred_element_type=_F32, precision=hp)
    Kf = jnp.dot(agg_uam, Wk[...], preferred_element_type=_F32, precision=hp)
    Kc = jnp.dot(ufnm, Wk[...], preferred_element_type=_F32, precision=hp)
    Vf = jnp.dot(agg_uam, Wv[...], preferred_element_type=_F32, precision=hp)
    Vc = jnp.dot(ufnm, Wv[...], preferred_element_type=_F32, precision=hp)
    s0 = jnp.dot(Q * Kf, GH, preferred_element_type=_F32, precision=hp) / dk
    s1 = jnp.dot(Q * Kc, GH, preferred_element_type=_F32, precision=hp) / dk
    m = jnp.maximum(s0, s1)
    e0 = jnp.exp(s0 - m)
    e1 = jnp.exp(s1 - m)
    w0 = e0 / (e0 + e1)
    w1 = 1.0 - w0
    att = (jnp.dot(w0, GHT, preferred_element_type=_F32, precision=hp) * Vf
           + jnp.dot(w1, GHT, preferred_element_type=_F32, precision=hp) * Vc)
    motif_msg = jnp.dot(att, Wo[...], preferred_element_type=_F32,
                        precision=hp) + bo[...]

    ufnfn_o[0:N_MOTIF, :] = _gru(motif_msg, ufnf_v, gmWih[...], gmWhh[...],
                                 gmbih[...], gmbhh[...])
    ufnfn_o[N_MOTIF:, :] = jnp.zeros((NM_RD - N_MOTIF, HID), _F32)


def _e1m(ufnf, aggM2, aggF2, cnt2, p):
    r1 = lambda a: a.reshape(1, -1)
    return pl.pallas_call(
        _e1m_body,
        out_shape=jax.ShapeDtypeStruct((NM_RD, HID), _F32),
    )(ufnf, aggM2, aggF2, cnt2,
      r1(p['ac_bias']), r1(p['mc_bias']),
      p['la_Wq'], p['la_Wk'], p['la_Wv'], p['la_Wo'], r1(p['la_bo']),
      p['gm_Wih'], p['gm_Whh'], r1(p['gm_bih']), r1(p['gm_bhh']))


def _e1a_body(uaf, aggA2, ac_bias, gaWih, gaWhh, gabih, gabhh, out):
    uam = aggA2[0] + aggA2[1] + ac_bias[...]
    out[...] = _gru(uam, uaf[...], gaWih[...], gaWhh[...], gabih[...],
                    gabhh[...])


def _e1a(uaf, aggA2, p, blk=2048):
    r1 = lambda a: a.reshape(1, -1)
    grid = NA_RD // blk
    return pl.pallas_call(
        _e1a_body,
        grid=(grid,),
        in_specs=[
            pl.BlockSpec((blk, HID), lambda i: (i, 0)),
            pl.BlockSpec((NC, blk, HID), lambda i: (0, i, 0)),
            pl.BlockSpec((1, HID), lambda i: (0, 0)),
            pl.BlockSpec((HID, 3 * HID), lambda i: (0, 0)),
            pl.BlockSpec((HID, 3 * HID), lambda i: (0, 0)),
            pl.BlockSpec((1, 3 * HID), lambda i: (0, 0)),
            pl.BlockSpec((1, 3 * HID), lambda i: (0, 0)),
        ],
        out_specs=pl.BlockSpec((blk, HID), lambda i: (i, 0)),
        out_shape=jax.ShapeDtypeStruct((NA_RD, HID), _F32),
    )(uaf, aggA2, r1(p['ac_bias']),
      p['ga_Wih'], p['ga_Whh'], r1(p['ga_bih']), r1(p['ga_bhh']))


def _readout_body(nfa, nfm, ab, mb,
                  asum_o, amax_o, acnt_o, msum_o, mmax_o, mcnt_o,
                  rows_v, idx_v, bsum_a, bmax_a, bcnt_a, bsum_m, bmax_m, bcnt_m,
                  slab_v, res_v, sh_list0, sh_list1, sh_list2, sh_list3,
                  sh_list4, sh_list5, sem):
    s = lax.axis_index("s")
    c = lax.axis_index("c")
    wid = s * NC + c
    neg = jnp.full((L,), -3.0e38, _F32)
    zero = jnp.zeros((L,), _F32)

    def init(i, _):
        bsum_a[i] = zero
        bmax_a[i] = neg
        bcnt_a[i] = zero
        bsum_m[i] = zero
        bmax_m[i] = neg
        bcnt_m[i] = zero
        return 0
    lax.fori_loop(0, GBUF, init, 0)

    one = jnp.ones((L,), _F32)

    pltpu.sync_copy(nfa.at[pl.ds(wid * RA, RA)], rows_v)
    pltpu.sync_copy(ab.at[pl.ds(wid * RA, RA)], idx_v)

    def arow(rb, _):
        r0 = rb * L
        gvec = idx_v[pl.ds(r0, L)]
        for j in range(L):
            g = gvec[j]
            row = rows_v[r0 + j]
            bsum_a[g] = bsum_a[g] + row
            bmax_a[g] = jnp.maximum(bmax_a[g], row)
            bcnt_a[g] = bcnt_a[g] + one
        return 0
    lax.fori_loop(0, RA // L, arow, 0)

    pltpu.sync_copy(nfm.at[pl.ds(wid * RM, RM)], rows_v.at[pl.ds(0, RM)])
    pltpu.sync_copy(mb.at[pl.ds(wid * RM, RM)], idx_v.at[pl.ds(0, RM)])

    def mrow(rb, _):
        r0 = rb * L
        gvec = idx_v[pl.ds(r0, L)]
        for j in range(L):
            g = gvec[j]
            row = rows_v[r0 + j]
            bsum_m[g] = bsum_m[g] + row
            bmax_m[g] = jnp.maximum(bmax_m[g], row)
            bcnt_m[g] = bcnt_m[g] + one
        return 0
    lax.fori_loop(0, RM // L, mrow, 0)

    shs = [sh_list0, sh_list1, sh_list2, sh_list3, sh_list4, sh_list5]
    bufs = [bsum_a, bmax_a, bcnt_a, bsum_m, bmax_m, bcnt_m]
    outs = [asum_o, amax_o, acnt_o, msum_o, mmax_o, mcnt_o]
    for sh, buf in zip(shs, bufs):
        pltpu.sync_copy(buf, sh.at[s])
    plsc.subcore_barrier()

    for bi, (sh, out, is_max) in enumerate(
            zip(shs, outs, [False, True, False, False, True, False])):
        descs = [pltpu.async_copy(sh.at[t, pl.ds(s * RG, RG)],
                                  slab_v.at[t], sem) for t in range(NS)]
        for d in descs:
            d.wait()

        def comb(r, _):
            acc = slab_v[0, r]
            for t in range(1, NS):
                if is_max:
                    acc = jnp.maximum(acc, slab_v[t, r])
                else:
                    acc = acc + slab_v[t, r]
            res_v[r] = acc
            return 0
        lax.fori_loop(0, RG, comb, 0)
        pltpu.sync_copy(res_v, out.at[c, pl.ds(s * RG, RG)])
        plsc.subcore_barrier()


def _sc_readout(nfa, nfm, ab_p, mb_p):
    out_t = jax.ShapeDtypeStruct((NC, GBUF, HID), _F32)
    k = functools.partial(
        pl.kernel,
        out_type=[out_t] * 6,
        mesh=plsc.VectorSubcoreMesh(core_axis_name="c", subcore_axis_name="s"),
        scratch_types=(
            [pltpu.VMEM((RA, HID), _F32), pltpu.VMEM((RA,), jnp.int32)]
            + [pltpu.VMEM((GBUF, HID), _F32)] * 6
            + [pltpu.VMEM((NS, RG, HID), _F32), pltpu.VMEM((RG, HID), _F32)]
            + [pltpu.VMEM_SHARED((NS, GBUF, HID), _F32)] * 6
            + [pltpu.SemaphoreType.DMA]),
        compiler_params=pltpu.CompilerParams(use_tc_tiling_on_sc=False),
    )(_readout_body)
    return k(nfa, nfm, ab_p, mb_p)


def _e2_body(asum2, amax2, acnt2, msum2, mmax2, mcnt2,
             cpW1, cpb1, cpW2, cpb2, cmW1, cmb1, cmW2, cmb2, cqW, cqb, out):
    asum = asum2[0, :G, :] + asum2[1, :G, :]
    amax = jnp.maximum(amax2[0, :G, :], amax2[1, :G, :])
    acnt = acnt2[0, :G, :] + acnt2[1, :G, :]
    msum = msum2[0, :G, :] + msum2[1, :G, :]
    mmax = jnp.maximum(mmax2[0, :G, :], mmax2[1, :G, :])
    mcnt = mcnt2[0, :G, :] + mcnt2[1, :G, :]
    amean = asum / jnp.maximum(acnt, 1.0)
    amaxf = jnp.where(acnt > 0, amax, 0.0)
    mmean = msum / jnp.maximum(mcnt, 1.0)
    mmaxf = jnp.where(mcnt > 0, mmax, 0.0)
    comb = jnp.concatenate([amean, amaxf, mmean, mmaxf], axis=1)
    rep = jnp.maximum(jnp.dot(comb, cpW1[...], preferred_element_type=_F32, precision=lax.Precision.HIGHEST)
                      + cpb1[...], 0.0)
    rep = jnp.dot(rep, cpW2[...], preferred_element_type=_F32, precision=lax.Precision.HIGHEST) + cpb2[...]
    h3 = jnp.maximum(jnp.dot(rep, cmW1[...], preferred_element_type=_F32, precision=lax.Precision.HIGHEST)
                     + cmb1[...], 0.0)
    lg = jnp.dot(h3, cmW2[...], preferred_element_type=_F32, precision=lax.Precision.HIGHEST) + cmb2[...]
    out[...] = jnp.dot(lg, cqW[...], preferred_element_type=_F32, precision=lax.Precision.HIGHEST) + cqb[...]


def _e2(rd6, p):
    r1 = lambda a: a.reshape(1, -1)
    return pl.pallas_call(
        _e2_body,
        out_shape=jax.ShapeDtypeStruct((G, 2), _F32),
    )(*rd6,
      p['cp_W1'], r1(p['cp_b1']), p['cp_W2'], r1(p['cp_b2']),
      p['cm_W1'], r1(p['cm_b1']), p['cm_W2'], r1(p['cm_b2']),
      p['cq_W'], r1(p['cq_b']))


def _pad_idx(idx, n, fill):
    return jnp.concatenate(
        [idx, jnp.full((n - idx.shape[0],), fill, jnp.int32)]).reshape(-1, CHUNK)


def kernel(af, bf, fnf, fef, atom_edge_index, motif_edge_index, a2f_index,
           atom_batch, motif_batch, params):
    p = params
    srcA2d = _pad_idx(atom_edge_index[0], EA_PAD, 0)
    dstA2d = _pad_idx(atom_edge_index[1], EA_PAD, NA_BUF - 1)
    srcM2d = _pad_idx(motif_edge_index[0], EM_PAD, 0)
    dstM2d = _pad_idx(motif_edge_index[1], EM_PAD, NM_BUF - 1)
    a2f2d = _pad_idx(a2f_index, NA_BUF, NM_BUF - 1)
    bf_p = jnp.concatenate([bf, jnp.zeros((EA_PAD - E_ATOM, bf.shape[1]), _F32)])
    fef_p = jnp.concatenate([fef, jnp.zeros((EM_PAD - E_MOTIF, fef.shape[1]), _F32)])

    uaf, ufnf = _encoders(af, fnf, p)
    xgA, xgM = _sc_gather(uaf, ufnf, srcA2d, srcM2d)
    msgA = _edge_messages(bf_p.T, xgA.T, p['ac_We'], p['ac_be'], 2048).T
    msgM = _edge_messages(fef_p.T, xgM.T, p['mc_We'], p['mc_be'], 2048).T
    aggA2, aggM2, aggF2, cnt2 = _sc_scatter(msgA, msgM, dstA2d, dstM2d, a2f2d)
    ufnfn = _e1m(ufnf, aggM2, aggF2, cnt2, p)
    uafn = _e1a(uaf, aggA2[:, :NA_RD, :], p)
    ab_p = jnp.concatenate(
        [atom_batch, jnp.full((NA_RD - N_ATOM,), G, jnp.int32)])
    mb_p = jnp.concatenate(
        [motif_batch, jnp.full((NM_RD - N_MOTIF,), G, jnp.int32)])
    rd6 = _sc_readout(uafn, ufnfn, ab_p, mb_p)
    return _e2(rd6, p)
